# baseline (speedup 1.0000x reference)
import jax
import jax.numpy as jnp
from jax import lax
from jax.experimental import pallas as pl
from jax.experimental.pallas import tpu as pltpu
from jax.experimental.pallas import tpu_sc as plsc

N = 100000
E = 6400000
IN = 4
HID = 16

NC = 2
NS = 16
LANES = 16

PER_TILE_N = 6272
N_PAD = NS * PER_TILE_N
ROWS = N_PAD // 128

E_PER_CORE = E // NC
E_PER_TILE = E_PER_CORE // NS
CHUNK = 4000
NUM_CHUNKS = E_PER_TILE // CHUNK

_mesh = plsc.VectorSubcoreMesh(
    core_axis_name="c", subcore_axis_name="s", num_cores=NC, num_subcores=NS
)

_f32 = jnp.float32
_i32 = jnp.int32


def _zero_fill(buf, n):
    zeros = jnp.zeros((LANES,), _f32)

    def body(i, _):
        buf[pl.ds(i * LANES, LANES)] = zeros
        return 0

    lax.fori_loop(0, n // LANES, body, 0)


def _scale_joint(valsp, ewwp):
    unroll = 5

    def body(k, _):
        for u in range(unroll):
            o = (k * unroll + u) * LANES
            sl = pl.ds(o, LANES)
            w = ewwp[sl]
            for vj in valsp:
                vj[sl] = vj[sl] * w
        return 0

    lax.fori_loop(0, CHUNK // (unroll * LANES), body, 0)


def _l1_packed_body(xs0, xs1, xs2, xs3, src_hbm, dst_hbm, ew_hbm, out_hbm,
                    srcv0, srcv1, dstv0, dstv1, eww0, eww1,
                    g01a, g01b, g23a, g23b,
                    v0a, v0b, v1a, v1b, v2a, v2b, v3a, v3b,
                    pbufA, pbufB, packbuf,
                    esem0, esem1, gsem0, gsem1, ssem0, ssem1,
                    t01_sp, t23_sp, a0, a1, a2, a3):
    xs_hbm = (xs0, xs1, xs2, xs3)
    srcv = (srcv0, srcv1)
    dstv = (dstv0, dstv1)
    eww = (eww0, eww1)
    g01 = (g01a, g01b)
    g23 = (g23a, g23b)
    vals = ((v0a, v1a, v2a, v3a), (v0b, v1b, v2b, v3b))
    esem = (esem0, esem1)
    gsem = (gsem0, gsem1)
    ssem = (ssem0, ssem1)
    tabs_sp = (t01_sp, t23_sp)
    accs_sp = (a0, a1, a2, a3)

    c = lax.axis_index("c")
    s = lax.axis_index("s")
    tile_off = s * PER_TILE_N
    tsl = pl.ds(tile_off, PER_TILE_N)

    for j, tab in enumerate(tabs_sp):
        pltpu.sync_copy(xs_hbm[2 * j].at[tsl], pbufA)
        pltpu.sync_copy(xs_hbm[2 * j + 1].at[tsl], pbufB)

        def packloop(k, _):
            sl = pl.ds(k * LANES, LANES)
            w = plsc.pack(pbufA[sl], pbufB[sl],
                          format=plsc.PackFormat.INTERLEAVED)
            packbuf[sl] = plsc.bitcast(w, _i32)
            return 0

        lax.fori_loop(0, PER_TILE_N // LANES, packloop, 0)
        pltpu.sync_copy(packbuf, tab.at[tsl])

    _zero_fill(pbufA, PER_TILE_N)
    for a in accs_sp:
        pltpu.sync_copy(pbufA, a.at[tsl])
    plsc.subcore_barrier()

    def edge_off(i):
        off = c * E_PER_CORE + s * E_PER_TILE + i * CHUNK
        return pl.ds(pl.multiple_of(off, 8), CHUNK)

    def fire_edges(i, p):
        esl = edge_off(i)
        pltpu.async_copy(src_hbm.at[esl], srcv[p], esem[p])
        pltpu.async_copy(dst_hbm.at[esl], dstv[p], esem[p])
        pltpu.async_copy(ew_hbm.at[esl], eww[p], esem[p])

    def drain_edges(i, p):
        esl = edge_off(i)
        pltpu.make_async_copy(src_hbm.at[esl], srcv[p], esem[p]).wait()
        pltpu.make_async_copy(dst_hbm.at[esl], dstv[p], esem[p]).wait()
        pltpu.make_async_copy(ew_hbm.at[esl], eww[p], esem[p]).wait()

    def drain_scatters(p):
        for j in range(IN):
            pltpu.make_async_copy(vals[p][j], accs_sp[j].at[dstv[p]],
                                  ssem[p]).wait()

    unroll = 5

    def scale_unpack(p):
        vp = vals[p]
        ep = eww[p]
        gp01 = g01[p]
        gp23 = g23[p]

        def body(k, _):
            for u in range(unroll):
                sl = pl.ds((k * unroll + u) * LANES, LANES)
                w = ep[sl]
                f0, f1 = plsc.unpack(plsc.bitcast(gp01[sl], jnp.bfloat16),
                                     format=plsc.PackFormat.INTERLEAVED,
                                     preferred_element_type=_f32)
                f2, f3 = plsc.unpack(plsc.bitcast(gp23[sl], jnp.bfloat16),
                                     format=plsc.PackFormat.INTERLEAVED,
                                     preferred_element_type=_f32)
                vp[0][sl] = f0 * w
                vp[1][sl] = f1 * w
                vp[2][sl] = f2 * w
                vp[3][sl] = f3 * w
            return 0

        lax.fori_loop(0, CHUNK // (unroll * LANES), body, 0)

    def fire_gathers(p):
        pltpu.async_copy(t01_sp.at[srcv[p]], g01[p], gsem[p])
        pltpu.async_copy(t23_sp.at[srcv[p]], g23[p], gsem[p])

    def wait_gathers(p):
        pltpu.make_async_copy(t01_sp.at[srcv[p]], g01[p], gsem[p]).wait()
        pltpu.make_async_copy(t23_sp.at[srcv[p]], g23[p], gsem[p]).wait()

    fire_edges(0, 0)
    drain_edges(0, 0)
    fire_gathers(0)

    def outer(o, _):
        for p in (0, 1):
            iv = o * 2 + p
            q = 1 - p

            @pl.when(jnp.logical_and(iv >= 1, iv + 1 < NUM_CHUNKS))
            def _():
                drain_scatters(q)

            @pl.when(iv + 1 < NUM_CHUNKS)
            def _():
                fire_edges(iv + 1, q)
                drain_edges(iv + 1, q)
                fire_gathers(q)

            wait_gathers(p)
            scale_unpack(p)
            for j in range(IN):
                pltpu.async_copy(vals[p][j], accs_sp[j].at[dstv[p]],
                                 ssem[p], add=True)
        return 0

    lax.fori_loop(0, NUM_CHUNKS // 2, outer, 0)
    drain_scatters(0)
    drain_scatters(1)
    plsc.subcore_barrier()
    for j, a in enumerate(accs_sp):
        pltpu.sync_copy(a.at[tsl], out_hbm.at[c, j, tsl])


_l1_packed_scratch = (
    [pltpu.VMEM((CHUNK,), _i32)] * 4
    + [pltpu.VMEM((CHUNK,), _f32)] * 2
    + [pltpu.VMEM((CHUNK,), _i32)] * 4
    + [pltpu.VMEM((CHUNK,), _f32)] * 8
    + [pltpu.VMEM((PER_TILE_N,), _f32)] * 2
    + [pltpu.VMEM((PER_TILE_N,), _i32)]
    + [pltpu.SemaphoreType.DMA] * 6
    + [pltpu.VMEM_SHARED((N_PAD,), _i32)] * 2
    + [pltpu.VMEM_SHARED((N_PAD,), _f32)] * 4
)


def _make_edge_pass(nf):
    def body(*refs):
        it = iter(refs)
        tabs_hbm = [next(it) for _ in range(nf)]
        src_hbm = next(it) if nf else None
        dst_hbm = next(it)
        ew_hbm = next(it)
        out_hbm = next(it)
        srcv = [next(it), next(it)] if nf else None
        dstv = [next(it), next(it)]
        eww = [next(it), next(it)]
        vals = [[next(it) for _ in range(nf)] for _ in range(2)]
        zbuf = next(it)
        esem = [next(it), next(it)]
        gsem = [next(it), next(it)] if nf else None
        ssem = [next(it), next(it)]
        tabs_sp = [next(it) for _ in range(nf)]
        accs_sp = [next(it) for _ in range(max(nf, 1))]

        c = lax.axis_index("c")
        s = lax.axis_index("s")
        tile_off = s * PER_TILE_N
        tsl = pl.ds(tile_off, PER_TILE_N)

        _zero_fill(zbuf, PER_TILE_N)
        for j in range(nf):
            pltpu.sync_copy(tabs_hbm[j].at[tsl], tabs_sp[j].at[tsl])
        for a in accs_sp:
            pltpu.sync_copy(zbuf, a.at[tsl])
        plsc.subcore_barrier()

        def edge_off(i):
            off = c * E_PER_CORE + s * E_PER_TILE + i * CHUNK
            return pl.ds(pl.multiple_of(off, 8), CHUNK)

        def fire_edges(i, p):
            esl = edge_off(i)
            if nf:
                pltpu.async_copy(src_hbm.at[esl], srcv[p], esem[p])
            pltpu.async_copy(dst_hbm.at[esl], dstv[p], esem[p])
            pltpu.async_copy(ew_hbm.at[esl], eww[p], esem[p])

        def drain_edges(i, p):
            esl = edge_off(i)
            if nf:
                pltpu.make_async_copy(src_hbm.at[esl], srcv[p], esem[p]).wait()
            pltpu.make_async_copy(dst_hbm.at[esl], dstv[p], esem[p]).wait()
            pltpu.make_async_copy(ew_hbm.at[esl], eww[p], esem[p]).wait()

        def fire_scatters(p):
            if nf:
                for j in range(nf):
                    pltpu.async_copy(vals[p][j], accs_sp[j].at[dstv[p]],
                                     ssem[p], add=True)
            else:
                pltpu.async_copy(eww[p], accs_sp[0].at[dstv[p]],
                                 ssem[p], add=True)

        def drain_scatters(p):
            if nf:
                for j in range(nf):
                    pltpu.make_async_copy(vals[p][j],
                                          accs_sp[j].at[dstv[p]],
                                          ssem[p]).wait()
            else:
                pltpu.make_async_copy(eww[p], accs_sp[0].at[dstv[p]],
                                      ssem[p]).wait()

        def fire_gathers(p):
            for j in range(nf):
                pltpu.async_copy(tabs_sp[j].at[srcv[p]], vals[p][j], gsem[p])

        def wait_gathers(p):
            for j in range(nf):
                pltpu.make_async_copy(tabs_sp[j].at[srcv[p]],
                                      vals[p][j], gsem[p]).wait()

        fire_edges(0, 0)
        if nf:
            drain_edges(0, 0)
            fire_gathers(0)

        def outer(o, _):
            for p in (0, 1):
                iv = o * 2 + p
                q = 1 - p
                if nf:
                    @pl.when(jnp.logical_and(iv >= 1, iv + 1 < NUM_CHUNKS))
                    def _():
                        drain_scatters(q)

                    @pl.when(iv + 1 < NUM_CHUNKS)
                    def _():
                        fire_edges(iv + 1, q)
                        drain_edges(iv + 1, q)
                        fire_gathers(q)

                    wait_gathers(p)
                    _scale_joint(vals[p], eww[p])
                else:
                    drain_edges(iv, p)

                    @pl.when(jnp.logical_and(iv >= 1, iv + 1 < NUM_CHUNKS))
                    def _():
                        drain_scatters(q)

                    @pl.when(iv + 1 < NUM_CHUNKS)
                    def _():
                        fire_edges(iv + 1, q)

                fire_scatters(p)
            return 0

        lax.fori_loop(0, NUM_CHUNKS // 2, outer, 0)
        drain_scatters(0)
        drain_scatters(1)
        plsc.subcore_barrier()
        for j, a in enumerate(accs_sp):
            if len(accs_sp) == 1:
                dst_slice = out_hbm.at[c, tsl]
            else:
                dst_slice = out_hbm.at[c, j, tsl]
            pltpu.sync_copy(a.at[tsl], dst_slice)

    return body


def _edge_pass_scratch(nf):
    sems = [pltpu.SemaphoreType.DMA] * (6 if nf else 4)
    return (
        ([pltpu.VMEM((CHUNK,), _i32)] * 2 if nf else [])
        + [pltpu.VMEM((CHUNK,), _i32)] * 2
        + [pltpu.VMEM((CHUNK,), _f32)] * 2
        + [pltpu.VMEM((CHUNK,), _f32)] * (2 * nf)
        + [pltpu.VMEM((PER_TILE_N,), _f32)]
        + sems
        + [pltpu.VMEM_SHARED((N_PAD,), _f32)] * nf
        + [pltpu.VMEM_SHARED((N_PAD,), _f32)] * max(nf, 1)
    )


_deg_call = pl.kernel(
    _make_edge_pass(0),
    out_type=jax.ShapeDtypeStruct((NC, N_PAD), _f32),
    mesh=_mesh,
    scratch_types=_edge_pass_scratch(0),
)

_l1_call = pl.kernel(
    _l1_packed_body,
    out_type=jax.ShapeDtypeStruct((NC, IN, N_PAD), _f32),
    mesh=_mesh,
    scratch_types=_l1_packed_scratch,
    compiler_params=pltpu.CompilerParams(needs_layout_passes=False),
)

_l2_call = pl.kernel(
    _make_edge_pass(1),
    out_type=jax.ShapeDtypeStruct((NC, N_PAD), _f32),
    mesh=_mesh,
    scratch_types=_edge_pass_scratch(1),
)


def _tcA(degp_ref, xT_ref, dinv_ref, xs_ref):
    deg = degp_ref[0] + degp_ref[1] + 1.0
    dinv = lax.rsqrt(deg)
    dinv_ref[...] = dinv
    for j in range(IN):
        xs_ref[j] = xT_ref[j] * dinv


def _tcB(acc1_ref, xs_ref, dinv_ref, W1_ref, b1_ref, W2_ref, ys_ref):
    dinv = dinv_ref[...]
    agg = [dinv * (acc1_ref[0, j] + acc1_ref[1, j] + xs_ref[j]) for j in range(IN)]
    y = jnp.zeros_like(dinv)
    for t in range(HID):
        h = b1_ref[t]
        for j in range(IN):
            h = h + agg[j] * W1_ref[j, t]
        y = y + jnp.maximum(h, 0.0) * W2_ref[t, 0]
    ys_ref[...] = y * dinv


def _tcC(acc2_ref, ys_ref, dinv_ref, b2_ref, out_ref):
    out_ref[...] = (
        dinv_ref[...] * (acc2_ref[0] + acc2_ref[1] + ys_ref[...]) + b2_ref[0]
    )


def _vm():
    return pl.BlockSpec(memory_space=pltpu.MemorySpace.VMEM)


def _sm():
    return pl.BlockSpec(memory_space=pltpu.MemorySpace.SMEM)


def kernel(x, edge_index, edge_weight, W1, b1, W2, b2):
    src = edge_index[0]
    dst = edge_index[1]
    ew = edge_weight

    degp = _deg_call(dst, ew)

    xT = jnp.pad(x, ((0, N_PAD - N), (0, 0))).T.reshape(IN, ROWS, 128)
    dinv, xs = pl.pallas_call(
        _tcA,
        out_shape=(
            jax.ShapeDtypeStruct((ROWS, 128), _f32),
            jax.ShapeDtypeStruct((IN, ROWS, 128), _f32),
        ),
        in_specs=[_vm(), _vm()],
        out_specs=(_vm(), _vm()),
    )(degp.reshape(NC, ROWS, 128), xT)

    xs_flat = xs.reshape(IN, N_PAD)
    acc1 = _l1_call(xs_flat[0], xs_flat[1], xs_flat[2], xs_flat[3], src, dst, ew)

    ys = pl.pallas_call(
        _tcB,
        out_shape=jax.ShapeDtypeStruct((ROWS, 128), _f32),
        in_specs=[_vm(), _vm(), _vm(), _sm(), _sm(), _sm()],
        out_specs=_vm(),
    )(acc1.reshape(NC, IN, ROWS, 128), xs, dinv, W1, b1, W2)

    acc2 = _l2_call(ys.reshape(N_PAD), src, dst, ew)

    out = pl.pallas_call(
        _tcC,
        out_shape=jax.ShapeDtypeStruct((ROWS, 128), _f32),
        in_specs=[_vm(), _vm(), _vm(), _sm()],
        out_specs=_vm(),
    )(acc2.reshape(NC, ROWS, 128), ys, dinv, b2)

    return out.reshape(-1)[:N]


# speedup vs baseline: 1.0671x; 1.0671x over previous
import jax
import jax.numpy as jnp
from jax import lax
from jax.experimental import pallas as pl
from jax.experimental.pallas import tpu as pltpu
from jax.experimental.pallas import tpu_sc as plsc

N = 100000
E = 6400000
IN = 4
HID = 16

NC = 2
NS = 16
LANES = 16

PER_TILE_N = 6272
N_PAD = NS * PER_TILE_N
ROWS = N_PAD // 128

E_PER_CORE = E // NC
E_PER_TILE = E_PER_CORE // NS
CHUNK = 4000
NUM_CHUNKS = E_PER_TILE // CHUNK

_mesh = plsc.VectorSubcoreMesh(
    core_axis_name="c", subcore_axis_name="s", num_cores=NC, num_subcores=NS
)

_f32 = jnp.float32
_i32 = jnp.int32


def _zero_fill(buf, n):
    zeros = jnp.zeros((LANES,), _f32)

    def body(i, _):
        buf[pl.ds(i * LANES, LANES)] = zeros
        return 0

    lax.fori_loop(0, n // LANES, body, 0)


def _scale_joint(valsp, ewwp):
    unroll = 5

    def body(k, _):
        for u in range(unroll):
            o = (k * unroll + u) * LANES
            sl = pl.ds(o, LANES)
            w = ewwp[sl]
            for vj in valsp:
                vj[sl] = vj[sl] * w
        return 0

    lax.fori_loop(0, CHUNK // (unroll * LANES), body, 0)


def _l1_packed_body(xs0, xs1, xs2, xs3, src_hbm, dst_hbm, ew_hbm, out_hbm,
                    srcv0, srcv1, dstv0, dstv1, eww0, eww1,
                    g01a, g01b, g23a, g23b,
                    v0a, v0b, v1a, v1b, v2a, v2b, v3a, v3b,
                    pbufA, pbufB, packbuf,
                    esem0, esem1, gsem0, gsem1, ssem0, ssem1,
                    t01_sp, t23_sp, a0, a1, a2, a3):
    xs_hbm = (xs0, xs1, xs2, xs3)
    srcv = (srcv0, srcv1)
    dstv = (dstv0, dstv1)
    eww = (eww0, eww1)
    g01 = (g01a, g01b)
    g23 = (g23a, g23b)
    vals = ((v0a, v1a, v2a, v3a), (v0b, v1b, v2b, v3b))
    esem = (esem0, esem1)
    gsem = (gsem0, gsem1)
    ssem = (ssem0, ssem1)
    tabs_sp = (t01_sp, t23_sp)
    accs_sp = (a0, a1, a2, a3)

    c = lax.axis_index("c")
    s = lax.axis_index("s")
    tile_off = s * PER_TILE_N
    tsl = pl.ds(tile_off, PER_TILE_N)

    for j, tab in enumerate(tabs_sp):
        pltpu.sync_copy(xs_hbm[2 * j].at[tsl], pbufA)
        pltpu.sync_copy(xs_hbm[2 * j + 1].at[tsl], pbufB)

        def packloop(k, _):
            sl = pl.ds(k * LANES, LANES)
            w = plsc.pack(pbufA[sl], pbufB[sl],
                          format=plsc.PackFormat.INTERLEAVED)
            packbuf[sl] = plsc.bitcast(w, _i32)
            return 0

        lax.fori_loop(0, PER_TILE_N // LANES, packloop, 0)
        pltpu.sync_copy(packbuf, tab.at[tsl])

    _zero_fill(pbufA, PER_TILE_N)
    for a in accs_sp:
        pltpu.sync_copy(pbufA, a.at[tsl])
    plsc.subcore_barrier()

    def edge_off(i):
        off = c * E_PER_CORE + s * E_PER_TILE + i * CHUNK
        return pl.ds(pl.multiple_of(off, 8), CHUNK)

    def fire_edges(i, p):
        esl = edge_off(i)
        pltpu.async_copy(src_hbm.at[esl], srcv[p], esem[p])
        pltpu.async_copy(dst_hbm.at[esl], dstv[p], esem[p])
        pltpu.async_copy(ew_hbm.at[esl], eww[p], esem[p])

    def drain_edges(i, p):
        esl = edge_off(i)
        pltpu.make_async_copy(src_hbm.at[esl], srcv[p], esem[p]).wait()
        pltpu.make_async_copy(dst_hbm.at[esl], dstv[p], esem[p]).wait()
        pltpu.make_async_copy(ew_hbm.at[esl], eww[p], esem[p]).wait()

    def drain_scatters(p):
        for j in range(IN):
            pltpu.make_async_copy(vals[p][j], accs_sp[j].at[dstv[p]],
                                  ssem[p]).wait()

    unroll = 5

    def scale_unpack(p):
        vp = vals[p]
        ep = eww[p]
        gp01 = g01[p]
        gp23 = g23[p]

        def body(k, _):
            for u in range(unroll):
                sl = pl.ds((k * unroll + u) * LANES, LANES)
                w = ep[sl]
                f0, f1 = plsc.unpack(plsc.bitcast(gp01[sl], jnp.bfloat16),
                                     format=plsc.PackFormat.INTERLEAVED,
                                     preferred_element_type=_f32)
                f2, f3 = plsc.unpack(plsc.bitcast(gp23[sl], jnp.bfloat16),
                                     format=plsc.PackFormat.INTERLEAVED,
                                     preferred_element_type=_f32)
                vp[0][sl] = f0 * w
                vp[1][sl] = f1 * w
                vp[2][sl] = f2 * w
                vp[3][sl] = f3 * w
            return 0

        lax.fori_loop(0, CHUNK // (unroll * LANES), body, 0)

    fire_edges(0, 0)

    def outer(o, _):
        for p in (0, 1):
            iv = o * 2 + p
            drain_edges(iv, p)
            pltpu.async_copy(t01_sp.at[srcv[p]], g01[p], gsem[p])
            pltpu.async_copy(t23_sp.at[srcv[p]], g23[p], gsem[p])

            @pl.when(jnp.logical_and(iv >= 1, iv + 1 < NUM_CHUNKS))
            def _():
                drain_scatters(1 - p)

            @pl.when(iv + 1 < NUM_CHUNKS)
            def _():
                fire_edges(iv + 1, 1 - p)

            pltpu.make_async_copy(t01_sp.at[srcv[p]], g01[p], gsem[p]).wait()
            pltpu.make_async_copy(t23_sp.at[srcv[p]], g23[p], gsem[p]).wait()
            scale_unpack(p)
            for j in range(IN):
                pltpu.async_copy(vals[p][j], accs_sp[j].at[dstv[p]],
                                 ssem[p], add=True)
        return 0

    lax.fori_loop(0, NUM_CHUNKS // 2, outer, 0)
    drain_scatters(0)
    drain_scatters(1)
    plsc.subcore_barrier()
    for j, a in enumerate(accs_sp):
        pltpu.sync_copy(a.at[tsl], out_hbm.at[c, j, tsl])


_l1_packed_scratch = (
    [pltpu.VMEM((CHUNK,), _i32)] * 4
    + [pltpu.VMEM((CHUNK,), _f32)] * 2
    + [pltpu.VMEM((CHUNK,), _i32)] * 4
    + [pltpu.VMEM((CHUNK,), _f32)] * 8
    + [pltpu.VMEM((PER_TILE_N,), _f32)] * 2
    + [pltpu.VMEM((PER_TILE_N,), _i32)]
    + [pltpu.SemaphoreType.DMA] * 6
    + [pltpu.VMEM_SHARED((N_PAD,), _i32)] * 2
    + [pltpu.VMEM_SHARED((N_PAD,), _f32)] * 4
)


def _make_edge_pass(nf):
    def body(*refs):
        it = iter(refs)
        tabs_hbm = [next(it) for _ in range(nf)]
        src_hbm = next(it) if nf else None
        dst_hbm = next(it)
        ew_hbm = next(it)
        out_hbm = next(it)
        srcv = [next(it), next(it)] if nf else None
        dstv = [next(it), next(it)]
        eww = [next(it), next(it)]
        vals = [[next(it) for _ in range(nf)] for _ in range(2)]
        zbuf = next(it)
        esem = [next(it), next(it)]
        gsem = [next(it), next(it)] if nf else None
        ssem = [next(it), next(it)]
        tabs_sp = [next(it) for _ in range(nf)]
        accs_sp = [next(it) for _ in range(max(nf, 1))]

        c = lax.axis_index("c")
        s = lax.axis_index("s")
        tile_off = s * PER_TILE_N
        tsl = pl.ds(tile_off, PER_TILE_N)

        _zero_fill(zbuf, PER_TILE_N)
        for j in range(nf):
            pltpu.sync_copy(tabs_hbm[j].at[tsl], tabs_sp[j].at[tsl])
        for a in accs_sp:
            pltpu.sync_copy(zbuf, a.at[tsl])
        plsc.subcore_barrier()

        def edge_off(i):
            off = c * E_PER_CORE + s * E_PER_TILE + i * CHUNK
            return pl.ds(pl.multiple_of(off, 8), CHUNK)

        def fire_edges(i, p):
            esl = edge_off(i)
            if nf:
                pltpu.async_copy(src_hbm.at[esl], srcv[p], esem[p])
            pltpu.async_copy(dst_hbm.at[esl], dstv[p], esem[p])
            pltpu.async_copy(ew_hbm.at[esl], eww[p], esem[p])

        def drain_edges(i, p):
            esl = edge_off(i)
            if nf:
                pltpu.make_async_copy(src_hbm.at[esl], srcv[p], esem[p]).wait()
            pltpu.make_async_copy(dst_hbm.at[esl], dstv[p], esem[p]).wait()
            pltpu.make_async_copy(ew_hbm.at[esl], eww[p], esem[p]).wait()

        def fire_scatters(p):
            if nf:
                for j in range(nf):
                    pltpu.async_copy(vals[p][j], accs_sp[j].at[dstv[p]],
                                     ssem[p], add=True)
            else:
                pltpu.async_copy(eww[p], accs_sp[0].at[dstv[p]],
                                 ssem[p], add=True)

        def drain_scatters(p):
            if nf:
                for j in range(nf):
                    pltpu.make_async_copy(vals[p][j],
                                          accs_sp[j].at[dstv[p]],
                                          ssem[p]).wait()
            else:
                pltpu.make_async_copy(eww[p], accs_sp[0].at[dstv[p]],
                                      ssem[p]).wait()

        fire_edges(0, 0)

        def outer(o, _):
            for p in (0, 1):
                iv = o * 2 + p
                drain_edges(iv, p)
                if nf:
                    for j in range(nf):
                        pltpu.async_copy(tabs_sp[j].at[srcv[p]],
                                         vals[p][j], gsem[p])

                @pl.when(jnp.logical_and(iv >= 1, iv + 1 < NUM_CHUNKS))
                def _():
                    drain_scatters(1 - p)

                @pl.when(iv + 1 < NUM_CHUNKS)
                def _():
                    fire_edges(iv + 1, 1 - p)

                if nf:
                    for j in range(nf):
                        pltpu.make_async_copy(tabs_sp[j].at[srcv[p]],
                                              vals[p][j], gsem[p]).wait()
                    _scale_joint(vals[p], eww[p])
                fire_scatters(p)
            return 0

        lax.fori_loop(0, NUM_CHUNKS // 2, outer, 0)
        drain_scatters(0)
        drain_scatters(1)
        plsc.subcore_barrier()
        for j, a in enumerate(accs_sp):
            if len(accs_sp) == 1:
                dst_slice = out_hbm.at[c, tsl]
            else:
                dst_slice = out_hbm.at[c, j, tsl]
            pltpu.sync_copy(a.at[tsl], dst_slice)

    return body


def _edge_pass_scratch(nf):
    sems = [pltpu.SemaphoreType.DMA] * (6 if nf else 4)
    return (
        ([pltpu.VMEM((CHUNK,), _i32)] * 2 if nf else [])
        + [pltpu.VMEM((CHUNK,), _i32)] * 2
        + [pltpu.VMEM((CHUNK,), _f32)] * 2
        + [pltpu.VMEM((CHUNK,), _f32)] * (2 * nf)
        + [pltpu.VMEM((PER_TILE_N,), _f32)]
        + sems
        + [pltpu.VMEM_SHARED((N_PAD,), _f32)] * nf
        + [pltpu.VMEM_SHARED((N_PAD,), _f32)] * max(nf, 1)
    )


_deg_call = pl.kernel(
    _make_edge_pass(0),
    out_type=jax.ShapeDtypeStruct((NC, N_PAD), _f32),
    mesh=_mesh,
    scratch_types=_edge_pass_scratch(0),
)

_l1_call = pl.kernel(
    _l1_packed_body,
    out_type=jax.ShapeDtypeStruct((NC, IN, N_PAD), _f32),
    mesh=_mesh,
    scratch_types=_l1_packed_scratch,
    compiler_params=pltpu.CompilerParams(needs_layout_passes=False),
)

_l2_call = pl.kernel(
    _make_edge_pass(1),
    out_type=jax.ShapeDtypeStruct((NC, N_PAD), _f32),
    mesh=_mesh,
    scratch_types=_edge_pass_scratch(1),
)


def _tcA(degp_ref, xT_ref, dinv_ref, xs_ref):
    deg = degp_ref[0] + degp_ref[1] + 1.0
    dinv = lax.rsqrt(deg)
    dinv_ref[...] = dinv
    for j in range(IN):
        xs_ref[j] = xT_ref[j] * dinv


def _tcB(acc1_ref, xs_ref, dinv_ref, W1_ref, b1_ref, W2_ref, ys_ref):
    dinv = dinv_ref[...]
    agg = [dinv * (acc1_ref[0, j] + acc1_ref[1, j] + xs_ref[j]) for j in range(IN)]
    y = jnp.zeros_like(dinv)
    for t in range(HID):
        h = b1_ref[t]
        for j in range(IN):
            h = h + agg[j] * W1_ref[j, t]
        y = y + jnp.maximum(h, 0.0) * W2_ref[t, 0]
    ys_ref[...] = y * dinv


def _tcC(acc2_ref, ys_ref, dinv_ref, b2_ref, out_ref):
    out_ref[...] = (
        dinv_ref[...] * (acc2_ref[0] + acc2_ref[1] + ys_ref[...]) + b2_ref[0]
    )


def _vm():
    return pl.BlockSpec(memory_space=pltpu.MemorySpace.VMEM)


def _sm():
    return pl.BlockSpec(memory_space=pltpu.MemorySpace.SMEM)


def kernel(x, edge_index, edge_weight, W1, b1, W2, b2):
    src = edge_index[0]
    dst = edge_index[1]
    ew = edge_weight

    degp = _deg_call(dst, ew)

    xT = jnp.pad(x, ((0, N_PAD - N), (0, 0))).T.reshape(IN, ROWS, 128)
    dinv, xs = pl.pallas_call(
        _tcA,
        out_shape=(
            jax.ShapeDtypeStruct((ROWS, 128), _f32),
            jax.ShapeDtypeStruct((IN, ROWS, 128), _f32),
        ),
        in_specs=[_vm(), _vm()],
        out_specs=(_vm(), _vm()),
    )(degp.reshape(NC, ROWS, 128), xT)

    xs_flat = xs.reshape(IN, N_PAD)
    acc1 = _l1_call(xs_flat[0], xs_flat[1], xs_flat[2], xs_flat[3], src, dst, ew)

    ys = pl.pallas_call(
        _tcB,
        out_shape=jax.ShapeDtypeStruct((ROWS, 128), _f32),
        in_specs=[_vm(), _vm(), _vm(), _sm(), _sm(), _sm()],
        out_specs=_vm(),
    )(acc1.reshape(NC, IN, ROWS, 128), xs, dinv, W1, b1, W2)

    acc2 = _l2_call(ys.reshape(N_PAD), src, dst, ew)

    out = pl.pallas_call(
        _tcC,
        out_shape=jax.ShapeDtypeStruct((ROWS, 128), _f32),
        in_specs=[_vm(), _vm(), _vm(), _sm()],
        out_specs=_vm(),
    )(acc2.reshape(NC, ROWS, 128), ys, dinv, b2)

    return out.reshape(-1)[:N]


# speedup vs baseline: 1.0877x; 1.0193x over previous
import jax
import jax.numpy as jnp
from jax import lax
from jax.experimental import pallas as pl
from jax.experimental.pallas import tpu as pltpu
from jax.experimental.pallas import tpu_sc as plsc

N = 100000
E = 6400000
IN = 4
HID = 16

NC = 2
NS = 16
LANES = 16

PER_TILE_N = 6272
N_PAD = NS * PER_TILE_N
ROWS = N_PAD // 128

E_PER_CORE = E // NC
E_PER_TILE = E_PER_CORE // NS
CHUNK = 4000
NUM_CHUNKS = E_PER_TILE // CHUNK

_mesh = plsc.VectorSubcoreMesh(
    core_axis_name="c", subcore_axis_name="s", num_cores=NC, num_subcores=NS
)

_f32 = jnp.float32
_i32 = jnp.int32


def _zero_fill(buf, n):
    zeros = jnp.zeros((LANES,), _f32)

    def body(i, _):
        buf[pl.ds(i * LANES, LANES)] = zeros
        return 0

    lax.fori_loop(0, n // LANES, body, 0)


def _scale_joint(valsp, ewwp, chunk):
    unroll = 5

    def body(k, _):
        for u in range(unroll):
            o = (k * unroll + u) * LANES
            sl = pl.ds(o, LANES)
            w = ewwp[sl]
            for vj in valsp:
                vj[sl] = vj[sl] * w
        return 0

    lax.fori_loop(0, chunk // (unroll * LANES), body, 0)


def _l1_packed_body(xs0, xs1, xs2, xs3, src_hbm, dst_hbm, ew_hbm, out_hbm,
                    srcv0, srcv1, dstv0, dstv1, eww0, eww1,
                    g01a, g01b, g23a, g23b,
                    v0a, v0b, v1a, v1b, v2a, v2b, v3a, v3b,
                    pbufA, pbufB, packbuf,
                    esem0, esem1, gsem0, gsem1, ssem0, ssem1,
                    t01_sp, t23_sp, a0, a1, a2, a3):
    xs_hbm = (xs0, xs1, xs2, xs3)
    srcv = (srcv0, srcv1)
    dstv = (dstv0, dstv1)
    eww = (eww0, eww1)
    g01 = (g01a, g01b)
    g23 = (g23a, g23b)
    vals = ((v0a, v1a, v2a, v3a), (v0b, v1b, v2b, v3b))
    esem = (esem0, esem1)
    gsem = (gsem0, gsem1)
    ssem = (ssem0, ssem1)
    tabs_sp = (t01_sp, t23_sp)
    accs_sp = (a0, a1, a2, a3)

    c = lax.axis_index("c")
    s = lax.axis_index("s")
    tile_off = s * PER_TILE_N
    tsl = pl.ds(tile_off, PER_TILE_N)

    for j, tab in enumerate(tabs_sp):
        pltpu.sync_copy(xs_hbm[2 * j].at[tsl], pbufA)
        pltpu.sync_copy(xs_hbm[2 * j + 1].at[tsl], pbufB)

        def packloop(k, _):
            sl = pl.ds(k * LANES, LANES)
            w = plsc.pack(pbufA[sl], pbufB[sl],
                          format=plsc.PackFormat.INTERLEAVED)
            packbuf[sl] = plsc.bitcast(w, _i32)
            return 0

        lax.fori_loop(0, PER_TILE_N // LANES, packloop, 0)
        pltpu.sync_copy(packbuf, tab.at[tsl])

    _zero_fill(pbufA, PER_TILE_N)
    for a in accs_sp:
        pltpu.sync_copy(pbufA, a.at[tsl])
    plsc.subcore_barrier()

    def edge_off(i):
        off = c * E_PER_CORE + s * E_PER_TILE + i * CHUNK
        return pl.ds(pl.multiple_of(off, 8), CHUNK)

    def fire_edges(i, p):
        esl = edge_off(i)
        pltpu.async_copy(src_hbm.at[esl], srcv[p], esem[p])
        pltpu.async_copy(dst_hbm.at[esl], dstv[p], esem[p])
        pltpu.async_copy(ew_hbm.at[esl], eww[p], esem[p])

    def drain_edges(i, p):
        esl = edge_off(i)
        pltpu.make_async_copy(src_hbm.at[esl], srcv[p], esem[p]).wait()
        pltpu.make_async_copy(dst_hbm.at[esl], dstv[p], esem[p]).wait()
        pltpu.make_async_copy(ew_hbm.at[esl], eww[p], esem[p]).wait()

    def drain_scatters(p):
        for j in range(IN):
            pltpu.make_async_copy(vals[p][j], accs_sp[j].at[dstv[p]],
                                  ssem[p]).wait()

    unroll = 5

    def scale_unpack(p):
        vp = vals[p]
        ep = eww[p]
        gp01 = g01[p]
        gp23 = g23[p]

        def body(k, _):
            for u in range(unroll):
                sl = pl.ds((k * unroll + u) * LANES, LANES)
                w = ep[sl]
                f0, f1 = plsc.unpack(plsc.bitcast(gp01[sl], jnp.bfloat16),
                                     format=plsc.PackFormat.INTERLEAVED,
                                     preferred_element_type=_f32)
                f2, f3 = plsc.unpack(plsc.bitcast(gp23[sl], jnp.bfloat16),
                                     format=plsc.PackFormat.INTERLEAVED,
                                     preferred_element_type=_f32)
                vp[0][sl] = f0 * w
                vp[1][sl] = f1 * w
                vp[2][sl] = f2 * w
                vp[3][sl] = f3 * w
            return 0

        lax.fori_loop(0, CHUNK // (unroll * LANES), body, 0)

    fire_edges(0, 0)

    def outer(o, _):
        for p in (0, 1):
            iv = o * 2 + p
            drain_edges(iv, p)
            pltpu.async_copy(t01_sp.at[srcv[p]], g01[p], gsem[p])
            pltpu.async_copy(t23_sp.at[srcv[p]], g23[p], gsem[p])

            @pl.when(jnp.logical_and(iv >= 1, iv + 1 < NUM_CHUNKS))
            def _():
                drain_scatters(1 - p)

            @pl.when(iv + 1 < NUM_CHUNKS)
            def _():
                fire_edges(iv + 1, 1 - p)

            pltpu.make_async_copy(t01_sp.at[srcv[p]], g01[p], gsem[p]).wait()
            pltpu.make_async_copy(t23_sp.at[srcv[p]], g23[p], gsem[p]).wait()
            scale_unpack(p)
            for j in range(IN):
                pltpu.async_copy(vals[p][j], accs_sp[j].at[dstv[p]],
                                 ssem[p], add=True)
        return 0

    lax.fori_loop(0, NUM_CHUNKS // 2, outer, 0)
    drain_scatters(0)
    drain_scatters(1)
    plsc.subcore_barrier()
    for j, a in enumerate(accs_sp):
        pltpu.sync_copy(a.at[tsl], out_hbm.at[c, j, tsl])


_l1_packed_scratch = (
    [pltpu.VMEM((CHUNK,), _i32)] * 4
    + [pltpu.VMEM((CHUNK,), _f32)] * 2
    + [pltpu.VMEM((CHUNK,), _i32)] * 4
    + [pltpu.VMEM((CHUNK,), _f32)] * 8
    + [pltpu.VMEM((PER_TILE_N,), _f32)] * 2
    + [pltpu.VMEM((PER_TILE_N,), _i32)]
    + [pltpu.SemaphoreType.DMA] * 6
    + [pltpu.VMEM_SHARED((N_PAD,), _i32)] * 2
    + [pltpu.VMEM_SHARED((N_PAD,), _f32)] * 4
)


def _make_edge_pass(nf, chunk=CHUNK):
    num_chunks = E_PER_TILE // chunk

    def body(*refs):
        it = iter(refs)
        tabs_hbm = [next(it) for _ in range(nf)]
        src_hbm = next(it) if nf else None
        dst_hbm = next(it)
        ew_hbm = next(it)
        out_hbm = next(it)
        srcv = [next(it), next(it)] if nf else None
        dstv = [next(it), next(it)]
        eww = [next(it), next(it)]
        vals = [[next(it) for _ in range(nf)] for _ in range(2)]
        zbuf = next(it)
        esem = [next(it), next(it)]
        gsem = [next(it), next(it)] if nf else None
        ssem = [next(it), next(it)]
        tabs_sp = [next(it) for _ in range(nf)]
        accs_sp = [next(it) for _ in range(max(nf, 1))]

        c = lax.axis_index("c")
        s = lax.axis_index("s")
        tile_off = s * PER_TILE_N
        tsl = pl.ds(tile_off, PER_TILE_N)

        _zero_fill(zbuf, PER_TILE_N)
        for j in range(nf):
            pltpu.sync_copy(tabs_hbm[j].at[tsl], tabs_sp[j].at[tsl])
        for a in accs_sp:
            pltpu.sync_copy(zbuf, a.at[tsl])
        plsc.subcore_barrier()

        def edge_off(i):
            off = c * E_PER_CORE + s * E_PER_TILE + i * chunk
            return pl.ds(pl.multiple_of(off, 8), chunk)

        def fire_edges(i, p):
            esl = edge_off(i)
            if nf:
                pltpu.async_copy(src_hbm.at[esl], srcv[p], esem[p])
            pltpu.async_copy(dst_hbm.at[esl], dstv[p], esem[p])
            pltpu.async_copy(ew_hbm.at[esl], eww[p], esem[p])

        def drain_edges(i, p):
            esl = edge_off(i)
            if nf:
                pltpu.make_async_copy(src_hbm.at[esl], srcv[p], esem[p]).wait()
            pltpu.make_async_copy(dst_hbm.at[esl], dstv[p], esem[p]).wait()
            pltpu.make_async_copy(ew_hbm.at[esl], eww[p], esem[p]).wait()

        def fire_scatters(p):
            if nf:
                for j in range(nf):
                    pltpu.async_copy(vals[p][j], accs_sp[j].at[dstv[p]],
                                     ssem[p], add=True)
            else:
                pltpu.async_copy(eww[p], accs_sp[0].at[dstv[p]],
                                 ssem[p], add=True)

        def drain_scatters(p):
            if nf:
                for j in range(nf):
                    pltpu.make_async_copy(vals[p][j],
                                          accs_sp[j].at[dstv[p]],
                                          ssem[p]).wait()
            else:
                pltpu.make_async_copy(eww[p], accs_sp[0].at[dstv[p]],
                                      ssem[p]).wait()

        fire_edges(0, 0)

        def outer(o, _):
            for p in (0, 1):
                iv = o * 2 + p
                drain_edges(iv, p)
                if nf:
                    for j in range(nf):
                        pltpu.async_copy(tabs_sp[j].at[srcv[p]],
                                         vals[p][j], gsem[p])

                @pl.when(jnp.logical_and(iv >= 1, iv + 1 < num_chunks))
                def _():
                    drain_scatters(1 - p)

                @pl.when(iv + 1 < num_chunks)
                def _():
                    fire_edges(iv + 1, 1 - p)

                if nf:
                    for j in range(nf):
                        pltpu.make_async_copy(tabs_sp[j].at[srcv[p]],
                                              vals[p][j], gsem[p]).wait()
                    _scale_joint(vals[p], eww[p], chunk)
                fire_scatters(p)
            return 0

        lax.fori_loop(0, num_chunks // 2, outer, 0)
        drain_scatters(0)
        drain_scatters(1)
        plsc.subcore_barrier()
        for j, a in enumerate(accs_sp):
            if len(accs_sp) == 1:
                dst_slice = out_hbm.at[c, tsl]
            else:
                dst_slice = out_hbm.at[c, j, tsl]
            pltpu.sync_copy(a.at[tsl], dst_slice)

    return body


def _edge_pass_scratch(nf, chunk=CHUNK):
    sems = [pltpu.SemaphoreType.DMA] * (6 if nf else 4)
    return (
        ([pltpu.VMEM((chunk,), _i32)] * 2 if nf else [])
        + [pltpu.VMEM((chunk,), _i32)] * 2
        + [pltpu.VMEM((chunk,), _f32)] * 2
        + [pltpu.VMEM((chunk,), _f32)] * (2 * nf)
        + [pltpu.VMEM((PER_TILE_N,), _f32)]
        + sems
        + [pltpu.VMEM_SHARED((N_PAD,), _f32)] * nf
        + [pltpu.VMEM_SHARED((N_PAD,), _f32)] * max(nf, 1)
    )


BIG_CHUNK = 10000

_deg_call = pl.kernel(
    _make_edge_pass(0, BIG_CHUNK),
    out_type=jax.ShapeDtypeStruct((NC, N_PAD), _f32),
    mesh=_mesh,
    scratch_types=_edge_pass_scratch(0, BIG_CHUNK),
)

_l1_call = pl.kernel(
    _l1_packed_body,
    out_type=jax.ShapeDtypeStruct((NC, IN, N_PAD), _f32),
    mesh=_mesh,
    scratch_types=_l1_packed_scratch,
    compiler_params=pltpu.CompilerParams(needs_layout_passes=False),
)

_l2_call = pl.kernel(
    _make_edge_pass(1, BIG_CHUNK),
    out_type=jax.ShapeDtypeStruct((NC, N_PAD), _f32),
    mesh=_mesh,
    scratch_types=_edge_pass_scratch(1, BIG_CHUNK),
)


def _tcA(degp_ref, xT_ref, dinv_ref, xs_ref):
    deg = degp_ref[0] + degp_ref[1] + 1.0
    dinv = lax.rsqrt(deg)
    dinv_ref[...] = dinv
    for j in range(IN):
        xs_ref[j] = xT_ref[j] * dinv


def _tcB(acc1_ref, xs_ref, dinv_ref, W1_ref, b1_ref, W2_ref, ys_ref):
    dinv = dinv_ref[...]
    agg = [dinv * (acc1_ref[0, j] + acc1_ref[1, j] + xs_ref[j]) for j in range(IN)]
    y = jnp.zeros_like(dinv)
    for t in range(HID):
        h = b1_ref[t]
        for j in range(IN):
            h = h + agg[j] * W1_ref[j, t]
        y = y + jnp.maximum(h, 0.0) * W2_ref[t, 0]
    ys_ref[...] = y * dinv


def _tcC(acc2_ref, ys_ref, dinv_ref, b2_ref, out_ref):
    out_ref[...] = (
        dinv_ref[...] * (acc2_ref[0] + acc2_ref[1] + ys_ref[...]) + b2_ref[0]
    )


def _vm():
    return pl.BlockSpec(memory_space=pltpu.MemorySpace.VMEM)


def _sm():
    return pl.BlockSpec(memory_space=pltpu.MemorySpace.SMEM)


def kernel(x, edge_index, edge_weight, W1, b1, W2, b2):
    src = edge_index[0]
    dst = edge_index[1]
    ew = edge_weight

    degp = _deg_call(dst, ew)

    xT = jnp.pad(x, ((0, N_PAD - N), (0, 0))).T.reshape(IN, ROWS, 128)
    dinv, xs = pl.pallas_call(
        _tcA,
        out_shape=(
            jax.ShapeDtypeStruct((ROWS, 128), _f32),
            jax.ShapeDtypeStruct((IN, ROWS, 128), _f32),
        ),
        in_specs=[_vm(), _vm()],
        out_specs=(_vm(), _vm()),
    )(degp.reshape(NC, ROWS, 128), xT)

    xs_flat = xs.reshape(IN, N_PAD)
    acc1 = _l1_call(xs_flat[0], xs_flat[1], xs_flat[2], xs_flat[3], src, dst, ew)

    ys = pl.pallas_call(
        _tcB,
        out_shape=jax.ShapeDtypeStruct((ROWS, 128), _f32),
        in_specs=[_vm(), _vm(), _vm(), _sm(), _sm(), _sm()],
        out_specs=_vm(),
    )(acc1.reshape(NC, IN, ROWS, 128), xs, dinv, W1, b1, W2)

    acc2 = _l2_call(ys.reshape(N_PAD), src, dst, ew)

    out = pl.pallas_call(
        _tcC,
        out_shape=jax.ShapeDtypeStruct((ROWS, 128), _f32),
        in_specs=[_vm(), _vm(), _vm(), _sm()],
        out_specs=_vm(),
    )(acc2.reshape(NC, ROWS, 128), ys, dinv, b2)

    return out.reshape(-1)[:N]
